# baseline (speedup 1.0000x reference)
import functools

import numpy as np
import jax
import jax.numpy as jnp
from jax.experimental import pallas as pl
from jax.experimental.pallas import tpu as pltpu


@functools.lru_cache(maxsize=None)
def _filter_consts(H: int, W: int, rate: float):
    n = np.arange(W)
    ang = 2.0 * np.pi * np.outer(n, n) / W
    Wc = np.exp(-1j * ang)
    Vc = np.exp(+1j * ang) / W
    cy, cx = H // 2, W // 2
    rh, rw = int(rate * cy), int(rate * cx)
    cols = np.arange(W)
    col_keep = (~((cols >= cx - rw) & (cols < cx + rw))).astype(np.float64)
    A = np.real((Wc * col_keep[None, :]) @ Vc).astype(np.float32)
    return A, cy - rh, cy + rh


def _row_filter_body(x_ref, b_ref, m_ref, o_ref):
    x = x_ref[...]
    y = jnp.dot(x.astype(jnp.bfloat16), b_ref[...],
                preferred_element_type=jnp.float32)
    o_ref[...] = x + m_ref[...] * y


def _filter_call(xf, Bmat, mask, TM, grid_n):
    W2 = Bmat.shape[0]
    return pl.pallas_call(
        _row_filter_body,
        out_shape=jax.ShapeDtypeStruct((xf.shape[0], W2), jnp.float32),
        grid=(grid_n,),
        in_specs=[
            pl.BlockSpec((TM, W2), lambda i: (i, 0)),
            pl.BlockSpec((W2, W2), lambda i: (0, 0)),
            pl.BlockSpec((TM, 1), lambda i: (0, 0)),
        ],
        out_specs=pl.BlockSpec((TM, W2), lambda i: (i, 0)),
        compiler_params=pltpu.CompilerParams(
            dimension_semantics=("arbitrary",),
            vmem_limit_bytes=64 * 2 ** 20),
    )(xf, Bmat, mask)


def kernel(x, rate: float = 0.95):
    B, C, H, W = x.shape
    A_np, lo, hi = _filter_consts(int(H), int(W), float(rate))
    AmI = A_np - np.eye(W, dtype=np.float32)
    M = B * C * H

    pair = (W == 128 and H % 2 == 0 and lo % 2 == 0 and hi % 2 == 0
            and M % 2 == 0)
    if pair:
        M2, W2, Hp = M // 2, 2 * W, H // 2
        Bmat = np.zeros((W2, W2), dtype=np.float32)
        Bmat[:W, :W] = AmI
        Bmat[W:, W:] = AmI
        xf = x.reshape(M2, W2).astype(jnp.float32)
        TM = 4096
        while M2 % TM != 0 or TM % Hp != 0:
            TM //= 2
        r = np.arange(TM) % Hp
        mask = ((2 * r >= lo) & (2 * r < hi)).astype(np.float32).reshape(TM, 1)
        out = _filter_call(xf, jnp.asarray(Bmat, dtype=jnp.bfloat16),
                           jnp.asarray(mask), TM, M2 // TM)
        return out.reshape(B, C, H, W)

    xf = x.reshape(M, W).astype(jnp.float32)
    TM = 8192
    while M % TM != 0 or TM % H != 0:
        TM //= 2
    r = np.arange(TM) % H
    mask = ((r >= lo) & (r < hi)).astype(np.float32).reshape(TM, 1)
    out = _filter_call(xf, jnp.asarray(AmI, dtype=jnp.bfloat16),
                       jnp.asarray(mask), TM, M // TM)
    return out.reshape(B, C, H, W)


# speedup vs baseline: 3.6327x; 3.6327x over previous
import functools

import numpy as np
import jax
import jax.numpy as jnp
from jax.experimental import pallas as pl
from jax.experimental.pallas import tpu as pltpu


@functools.lru_cache(maxsize=None)
def _filter_consts(H: int, W: int, rate: float):
    n = np.arange(W)
    ang = 2.0 * np.pi * np.outer(n, n) / W
    Wc = np.exp(-1j * ang)
    Vc = np.exp(+1j * ang) / W
    cy, cx = H // 2, W // 2
    rh, rw = int(rate * cy), int(rate * cx)
    cols = np.arange(W)
    col_keep = (~((cols >= cx - rw) & (cols < cx + rw))).astype(np.float64)
    A = np.real((Wc * col_keep[None, :]) @ Vc).astype(np.float32)
    return A, cy - rh, cy + rh


def _row_filter_body(x_ref, b_ref, m_ref, o_ref):
    x = x_ref[...]
    y = jnp.dot(x.astype(jnp.bfloat16), b_ref[...],
                preferred_element_type=jnp.float32)
    o_ref[...] = x + m_ref[...] * y


def kernel(x, rate: float = 0.95):
    B, C, H, W = x.shape
    A_np, lo, hi = _filter_consts(int(H), int(W), float(rate))
    AmI = jnp.asarray(A_np - np.eye(W, dtype=np.float32), dtype=jnp.bfloat16)

    M = B * C * H
    xf = x.reshape(M, W).astype(jnp.float32)

    TM = 8192
    while M % TM != 0 or TM % H != 0:
        TM //= 2

    r = np.arange(TM) % H
    mask = jnp.asarray(((r >= lo) & (r < hi)).reshape(TM, 1)
                       .astype(np.float32))

    out = pl.pallas_call(
        _row_filter_body,
        out_shape=jax.ShapeDtypeStruct((M, W), jnp.float32),
        grid=(M // TM,),
        in_specs=[
            pl.BlockSpec((TM, W), lambda i: (i, 0)),
            pl.BlockSpec((W, W), lambda i: (0, 0)),
            pl.BlockSpec((TM, 1), lambda i: (0, 0)),
        ],
        out_specs=pl.BlockSpec((TM, W), lambda i: (i, 0)),
        compiler_params=pltpu.CompilerParams(
            dimension_semantics=("arbitrary",),
            vmem_limit_bytes=64 * 2 ** 20),
    )(xf, AmI, mask)

    return out.reshape(B, C, H, W)


# speedup vs baseline: 3.6410x; 1.0023x over previous
import functools

import numpy as np
import jax
import jax.numpy as jnp
from jax.experimental import pallas as pl
from jax.experimental.pallas import tpu as pltpu


@functools.lru_cache(maxsize=None)
def _filter_consts(H: int, W: int, rate: float):
    n = np.arange(W)
    ang = 2.0 * np.pi * np.outer(n, n) / W
    Wc = np.exp(-1j * ang)
    Vc = np.exp(+1j * ang) / W
    cy, cx = H // 2, W // 2
    rh, rw = int(rate * cy), int(rate * cx)
    cols = np.arange(W)
    col_keep = (~((cols >= cx - rw) & (cols < cx + rw))).astype(np.float64)
    A = np.real((Wc * col_keep[None, :]) @ Vc).astype(np.float32)
    return A, cy - rh, cy + rh


def _row_filter_body(x_ref, b_ref, m_ref, o_ref):
    x = x_ref[...]
    y = jnp.dot(x.astype(jnp.bfloat16), b_ref[...],
                preferred_element_type=jnp.float32)
    o_ref[...] = jnp.where(m_ref[...] != 0, y, x)


def kernel(x, rate: float = 0.95):
    B, C, H, W = x.shape
    A_np, lo, hi = _filter_consts(int(H), int(W), float(rate))
    AmI = jnp.asarray(A_np, dtype=jnp.bfloat16)

    M = B * C * H
    xf = x.reshape(M, W).astype(jnp.float32)

    TM = 8192
    while M % TM != 0 or TM % H != 0:
        TM //= 2

    r = np.arange(TM) % H
    mask = jnp.asarray(((r >= lo) & (r < hi)).reshape(TM, 1)
                       .astype(np.float32))

    out = pl.pallas_call(
        _row_filter_body,
        out_shape=jax.ShapeDtypeStruct((M, W), jnp.float32),
        grid=(M // TM,),
        in_specs=[
            pl.BlockSpec((TM, W), lambda i: (i, 0)),
            pl.BlockSpec((W, W), lambda i: (0, 0)),
            pl.BlockSpec((TM, 1), lambda i: (0, 0)),
        ],
        out_specs=pl.BlockSpec((TM, W), lambda i: (i, 0)),
        compiler_params=pltpu.CompilerParams(
            dimension_semantics=("arbitrary",),
            vmem_limit_bytes=64 * 2 ** 20),
    )(xf, AmI, mask)

    return out.reshape(B, C, H, W)
